# baseline (speedup 1.0000x reference)
import functools
import jax
import jax.numpy as jnp
from jax.experimental import pallas as pl
from jax.experimental.pallas import tpu as pltpu


def _body(batch_ref, x_ref, w_ref, bias_ref, o_ref, psum_ref, oh_ref, *,
          tn, ch, nb, c, n_tiles):
    k = pl.program_id(0)

    @pl.when(k == 0)
    def _init():
        psum_ref[...] = jnp.zeros_like(psum_ref)

    gid = jax.lax.broadcasted_iota(jnp.int32, (ch, nb), 1)
    ones = jnp.ones((tn, 8), jnp.bfloat16)
    seg_col = jnp.transpose(batch_ref[0])
    for j in range(tn // ch):
        sc = seg_col[j * ch:(j + 1) * ch, :]
        oh_ref[pl.ds(j * ch, ch), :] = (sc == gid).astype(jnp.bfloat16)

    aug = jnp.concatenate([x_ref[...].astype(jnp.bfloat16), ones], axis=1)
    psum_ref[...] += jax.lax.dot_general(
        aug, oh_ref[...],
        dimension_numbers=(((0,), (0,)), ((), ())),
        preferred_element_type=jnp.float32)

    @pl.when(k == n_tiles - 1)
    def _finalize():
        s = psum_ref[...]
        pooled = s[:c, :] / jnp.maximum(s[c:c + 1, :], 1.0)
        y = jax.lax.dot_general(
            pooled, w_ref[...],
            dimension_numbers=(((0,), (1,)), ((), ())),
            preferred_element_type=jnp.float32)
        o_ref[...] = jnp.maximum(y + bias_ref[...], 0.0)


def _mean_pool_mlp(x, batch, weight, bias, num_graphs, tn, ch):
    n, c = x.shape
    h = weight.shape[0]
    assert n % tn == 0 and tn % ch == 0
    n_tiles = n // tn

    batch3 = batch.astype(jnp.int32).reshape(n_tiles, 1, tn)
    bias2 = bias.astype(jnp.float32).reshape(1, h)
    w = weight.astype(jnp.float32)

    out = pl.pallas_call(
        functools.partial(_body, tn=tn, ch=ch, nb=num_graphs, c=c,
                          n_tiles=n_tiles),
        out_shape=jax.ShapeDtypeStruct((num_graphs, h), jnp.float32),
        grid=(n_tiles,),
        in_specs=[
            pl.BlockSpec((1, 1, tn), lambda k: (k, 0, 0)),
            pl.BlockSpec((tn, c), lambda k: (k, 0)),
            pl.BlockSpec((h, c), lambda k: (0, 0)),
            pl.BlockSpec((1, h), lambda k: (0, 0)),
        ],
        out_specs=pl.BlockSpec((num_graphs, h), lambda k: (0, 0)),
        scratch_shapes=[pltpu.VMEM((c + 8, num_graphs), jnp.float32),
                        pltpu.VMEM((tn, num_graphs), jnp.bfloat16)],
        compiler_params=pltpu.CompilerParams(
            dimension_semantics=("arbitrary",),
            vmem_limit_bytes=56 * 1024 * 1024),
    )(batch3, x, w, bias2)
    return out


def kernel(x, batch, weight, bias):
    return _mean_pool_mlp(x, batch, weight, bias, 1024, 8192, 2048)


# speedup vs baseline: 1.0622x; 1.0622x over previous
import functools
import jax
import jax.numpy as jnp
from jax.experimental import pallas as pl
from jax.experimental.pallas import tpu as pltpu


def _body(batch_ref, x_ref, w_ref, bias_ref, o_ref, psum_ref, psum2_ref, *,
          tn, ch, nb, c, n_tiles):
    k = pl.program_id(0)

    @pl.when(k == 0)
    def _init():
        psum_ref[...] = jnp.zeros_like(psum_ref)
        psum2_ref[...] = jnp.zeros_like(psum2_ref)

    gid = jax.lax.broadcasted_iota(jnp.int32, (ch, nb), 1)
    ones = jnp.ones((ch, 8), jnp.bfloat16)
    seg_col = jnp.transpose(batch_ref[0])
    for j in range(tn // ch):
        xb = x_ref[pl.ds(j * ch, ch), :].astype(jnp.bfloat16)
        aug = jnp.concatenate([xb, ones], axis=1)
        sc = seg_col[j * ch:(j + 1) * ch, :]
        oh = (sc == gid).astype(jnp.bfloat16)
        acc = psum_ref if j % 2 == 0 else psum2_ref
        acc[...] += jax.lax.dot_general(
            aug, oh,
            dimension_numbers=(((0,), (0,)), ((), ())),
            preferred_element_type=jnp.float32)

    @pl.when(k == n_tiles - 1)
    def _finalize():
        s = psum_ref[...] + psum2_ref[...]
        pooled = s[:c, :] / jnp.maximum(s[c:c + 1, :], 1.0)
        y = jax.lax.dot_general(
            pooled, w_ref[...],
            dimension_numbers=(((0,), (1,)), ((), ())),
            preferred_element_type=jnp.float32)
        o_ref[...] = jnp.maximum(y + bias_ref[...], 0.0)


def _mean_pool_mlp(x, batch, weight, bias, num_graphs, tn, ch):
    n, c = x.shape
    h = weight.shape[0]
    assert n % tn == 0 and tn % ch == 0
    n_tiles = n // tn

    batch3 = batch.astype(jnp.int32).reshape(n_tiles, 1, tn)
    bias2 = bias.astype(jnp.float32).reshape(1, h)
    w = weight.astype(jnp.float32)

    out = pl.pallas_call(
        functools.partial(_body, tn=tn, ch=ch, nb=num_graphs, c=c,
                          n_tiles=n_tiles),
        out_shape=jax.ShapeDtypeStruct((num_graphs, h), jnp.float32),
        grid=(n_tiles,),
        in_specs=[
            pl.BlockSpec((1, 1, tn), lambda k: (k, 0, 0)),
            pl.BlockSpec((tn, c), lambda k: (k, 0)),
            pl.BlockSpec((h, c), lambda k: (0, 0)),
            pl.BlockSpec((1, h), lambda k: (0, 0)),
        ],
        out_specs=pl.BlockSpec((num_graphs, h), lambda k: (0, 0)),
        scratch_shapes=[pltpu.VMEM((c + 8, num_graphs), jnp.float32),
                        pltpu.VMEM((c + 8, num_graphs), jnp.float32)],
        compiler_params=pltpu.CompilerParams(
            dimension_semantics=("arbitrary",),
            vmem_limit_bytes=56 * 1024 * 1024),
    )(batch3, x, w, bias2)
    return out


def kernel(x, batch, weight, bias):
    return _mean_pool_mlp(x, batch, weight, bias, 1024, 8192, 2048)


# speedup vs baseline: 1.1136x; 1.0484x over previous
import functools
import jax
import jax.numpy as jnp
from jax.experimental import pallas as pl
from jax.experimental.pallas import tpu as pltpu


def _body(batch_ref, x_ref, w_ref, bias_ref, o_ref, psum_ref, psum2_ref, *,
          tn, ch, nb, c, n_tiles):
    k = pl.program_id(0)

    @pl.when(k == 0)
    def _init():
        psum_ref[...] = jnp.zeros_like(psum_ref)
        psum2_ref[...] = jnp.zeros_like(psum2_ref)

    gid = jax.lax.broadcasted_iota(jnp.int32, (ch, nb), 1)
    ones = jnp.ones((ch, 8), jnp.bfloat16)
    seg_col = jnp.transpose(batch_ref[0])
    for j in range(tn // ch):
        xb = x_ref[pl.ds(j * ch, ch), :].astype(jnp.bfloat16)
        aug = jnp.concatenate([xb, ones], axis=1)
        sc = seg_col[j * ch:(j + 1) * ch, :]
        oh = (sc == gid).astype(jnp.bfloat16)
        acc = psum_ref if j % 2 == 0 else psum2_ref
        acc[...] += jax.lax.dot_general(
            aug, oh,
            dimension_numbers=(((0,), (0,)), ((), ())),
            preferred_element_type=jnp.float32)

    @pl.when(k == n_tiles - 1)
    def _finalize():
        s = psum_ref[...] + psum2_ref[...]
        pooled = s[:c, :] / jnp.maximum(s[c:c + 1, :], 1.0)
        y = jax.lax.dot_general(
            pooled, w_ref[...],
            dimension_numbers=(((0,), (1,)), ((), ())),
            preferred_element_type=jnp.float32)
        o_ref[...] = jnp.maximum(y + bias_ref[...], 0.0)


def _mean_pool_mlp(x, batch, weight, bias, num_graphs, tn, ch):
    n, c = x.shape
    h = weight.shape[0]
    assert n % tn == 0 and tn % ch == 0
    n_tiles = n // tn

    batch3 = batch.astype(jnp.int32).reshape(n_tiles, 1, tn)
    bias2 = bias.astype(jnp.float32).reshape(1, h)
    w = weight.astype(jnp.float32)

    out = pl.pallas_call(
        functools.partial(_body, tn=tn, ch=ch, nb=num_graphs, c=c,
                          n_tiles=n_tiles),
        out_shape=jax.ShapeDtypeStruct((num_graphs, h), jnp.float32),
        grid=(n_tiles,),
        in_specs=[
            pl.BlockSpec((1, 1, tn), lambda k: (k, 0, 0)),
            pl.BlockSpec((tn, c), lambda k: (k, 0)),
            pl.BlockSpec((h, c), lambda k: (0, 0)),
            pl.BlockSpec((1, h), lambda k: (0, 0)),
        ],
        out_specs=pl.BlockSpec((num_graphs, h), lambda k: (0, 0)),
        scratch_shapes=[pltpu.VMEM((c + 8, num_graphs), jnp.float32),
                        pltpu.VMEM((c + 8, num_graphs), jnp.float32)],
        compiler_params=pltpu.CompilerParams(
            dimension_semantics=("arbitrary",),
            vmem_limit_bytes=56 * 1024 * 1024),
    )(batch3, x, w, bias2)
    return out


def kernel(x, batch, weight, bias):
    return _mean_pool_mlp(x, batch, weight, bias, 1024, 16384, 2048)
